# Initial kernel scaffold; baseline (speedup 1.0000x reference)
import jax
import jax.numpy as jnp
from jax.experimental import pallas as pl


def kernel(tokens, table):
    raise NotImplementedError("write your pallas kernel here")


# speedup vs baseline: 1.4271x; 1.4271x over previous
import functools
import math

import jax
import jax.numpy as jnp
from jax import lax
from jax.experimental import pallas as pl
from jax.experimental.pallas import tpu as pltpu
from jax.experimental.pallas import tpu_sc as plsc

VOCAB_SIZE = 100000
EMB_DIM = 512
NUM_CORES = 2
NUM_SUBCORES = 16
NUM_WORKERS = NUM_CORES * NUM_SUBCORES
LANES = 16
SCALE = math.sqrt(float(EMB_DIM))

CHUNK = 64
VECS_PER_ROW = EMB_DIM // LANES


def _emb_body(tokens_hbm, table_hbm, out_hbm, idx_v, buf0, buf1, sem0, sem1):
    n_chunks = tokens_hbm.shape[1]
    b_per_w = n_chunks * CHUNK
    wid = lax.axis_index("s") * NUM_CORES + lax.axis_index("c")
    base = wid * b_per_w

    pltpu.sync_copy(tokens_hbm.at[wid], idx_v)

    bufs = (buf0, buf1)
    sems = (sem0, sem1)
    scale_vec = jnp.full((LANES,), SCALE, jnp.float32)

    descs = [None] * n_chunks
    descs[0] = pltpu.async_copy(table_hbm.at[idx_v.at[0]], bufs[0], sems[0])
    for c in range(n_chunks):
        if c + 1 < n_chunks:
            descs[c + 1] = pltpu.async_copy(
                table_hbm.at[idx_v.at[c + 1]], bufs[(c + 1) % 2], sems[(c + 1) % 2]
            )
        descs[c].wait()
        buf = bufs[c % 2]

        @pl.loop(0, CHUNK)
        def _scale_row(r, buf=buf):
            for k in range(VECS_PER_ROW):
                sl = pl.ds(k * LANES, LANES)
                buf[r, sl] = buf[r, sl] * scale_vec

        pltpu.sync_copy(buf, out_hbm.at[pl.ds(base + c * CHUNK, CHUNK)])


@jax.jit
def _emb_lookup(tokens_grouped, table):
    n_tokens = tokens_grouped.shape[0] * tokens_grouped.shape[1] * tokens_grouped.shape[2]
    n_chunks = tokens_grouped.shape[1]
    mesh = plsc.VectorSubcoreMesh(
        core_axis_name="c", subcore_axis_name="s",
        num_cores=NUM_CORES, num_subcores=NUM_SUBCORES,
    )
    return pl.kernel(
        _emb_body,
        out_type=jax.ShapeDtypeStruct((n_tokens, EMB_DIM), jnp.float32),
        mesh=mesh,
        scratch_types=[
            pltpu.VMEM((n_chunks, CHUNK), jnp.int32),
            pltpu.VMEM((CHUNK, EMB_DIM), jnp.float32),
            pltpu.VMEM((CHUNK, EMB_DIM), jnp.float32),
            pltpu.SemaphoreType.DMA,
            pltpu.SemaphoreType.DMA,
        ],
    )(tokens_grouped, table)


def kernel(tokens, table):
    b, s = tokens.shape
    n = b * s
    per_w = n // NUM_WORKERS
    tokens_grouped = jnp.reshape(
        tokens.astype(jnp.int32), (NUM_WORKERS, per_w // CHUNK, CHUNK)
    )
    out = _emb_lookup(tokens_grouped, table)
    return jnp.reshape(out, (b, s, EMB_DIM))


# speedup vs baseline: 1.4448x; 1.0124x over previous
import functools
import math

import jax
import jax.numpy as jnp
from jax import lax
from jax.experimental import pallas as pl
from jax.experimental.pallas import tpu as pltpu
from jax.experimental.pallas import tpu_sc as plsc

VOCAB_SIZE = 100000
EMB_DIM = 512
NUM_CORES = 2
NUM_SUBCORES = 16
NUM_WORKERS = NUM_CORES * NUM_SUBCORES
LANES = 16
SCALE = math.sqrt(float(EMB_DIM))

CHUNK = 64
VECS_PER_ROW = EMB_DIM // LANES


NBUF = 3


def _emb_body(tokens_hbm, table_hbm, out_hbm, idx_v,
              buf0, buf1, buf2, gs0, gs1, gs2, ss0, ss1, ss2):
    n_chunks = tokens_hbm.shape[1]
    b_per_w = n_chunks * CHUNK
    wid = lax.axis_index("s") * NUM_CORES + lax.axis_index("c")
    base = wid * b_per_w

    pltpu.sync_copy(tokens_hbm.at[wid], idx_v)

    bufs = (buf0, buf1, buf2)
    gsems = (gs0, gs1, gs2)
    ssems = (ss0, ss1, ss2)
    scale_vec = jnp.full((LANES,), SCALE, jnp.float32)

    def gather(c):
        return pltpu.async_copy(
            table_hbm.at[idx_v.at[c]], bufs[c % NBUF], gsems[c % NBUF]
        )

    gds = [None] * n_chunks
    sds = [None] * n_chunks
    for c in range(min(NBUF - 1, n_chunks)):
        gds[c] = gather(c)
    for c in range(n_chunks):
        gds[c].wait()
        buf = bufs[c % NBUF]

        @pl.loop(0, CHUNK)
        def _scale_row(r, buf=buf):
            for k in range(VECS_PER_ROW):
                sl = pl.ds(k * LANES, LANES)
                buf[r, sl] = buf[r, sl] * scale_vec

        sds[c] = pltpu.async_copy(
            buf, out_hbm.at[pl.ds(base + c * CHUNK, CHUNK)], ssems[c % NBUF]
        )
        nc = c + NBUF - 1
        if nc < n_chunks:
            if nc >= NBUF:
                sds[nc - NBUF].wait()
            gds[nc] = gather(nc)
    for c in range(max(0, n_chunks - NBUF), n_chunks):
        sds[c].wait()


@jax.jit
def _emb_lookup(tokens_grouped, table):
    n_tokens = tokens_grouped.shape[0] * tokens_grouped.shape[1] * tokens_grouped.shape[2]
    n_chunks = tokens_grouped.shape[1]
    mesh = plsc.VectorSubcoreMesh(
        core_axis_name="c", subcore_axis_name="s",
        num_cores=NUM_CORES, num_subcores=NUM_SUBCORES,
    )
    return pl.kernel(
        _emb_body,
        out_type=jax.ShapeDtypeStruct((n_tokens, EMB_DIM), jnp.float32),
        mesh=mesh,
        scratch_types=[
            pltpu.VMEM((n_chunks, CHUNK), jnp.int32),
            pltpu.VMEM((CHUNK, EMB_DIM), jnp.float32),
            pltpu.VMEM((CHUNK, EMB_DIM), jnp.float32),
            pltpu.VMEM((CHUNK, EMB_DIM), jnp.float32),
            pltpu.SemaphoreType.DMA,
            pltpu.SemaphoreType.DMA,
            pltpu.SemaphoreType.DMA,
            pltpu.SemaphoreType.DMA,
            pltpu.SemaphoreType.DMA,
            pltpu.SemaphoreType.DMA,
        ],
    )(tokens_grouped, table)


def kernel(tokens, table):
    b, s = tokens.shape
    n = b * s
    per_w = n // NUM_WORKERS
    tokens_grouped = jnp.reshape(
        tokens.astype(jnp.int32), (NUM_WORKERS, per_w // CHUNK, CHUNK)
    )
    out = _emb_lookup(tokens_grouped, table)
    return jnp.reshape(out, (b, s, EMB_DIM))
